# baseline (speedup 1.0000x reference)
import functools

import numpy as np

import jax
import jax.numpy as jnp
from jax import lax
from jax.experimental import pallas as pl
from jax.experimental.pallas import tpu as pltpu
from jax.experimental.pallas import tpu_sc as plsc

_G = 33
_S = 2
_NN = _G * _G
_NROW = 1152
_NPAD = 1152
_NL = 8
_C = 128


_OFFS = (_S, _S * _G, _S * _G + _S, _S * _G - _S)
_PADL = 80


def _edge_tables():
    valid = np.zeros((4, _NPAD), np.float32)
    deg = np.zeros(_NPAD, np.float32)
    for i in range(_G):
        for j in range(_G):
            cur = i * _G + j
            dsts = []
            if j < _G - _S:
                dsts.append(cur + _S)
            if i < _G - _S:
                dsts.append(cur + _S * _G)
            if j < _G - _S and i < _G - _S:
                dsts.append(cur + _S * _G + _S)
            if j > _S and i < _G - _S:
                dsts.append(cur + _S * _G - _S)
            for d in dsts:
                valid[_OFFS.index(d - cur), d] = 1.0
                deg[d] += 1.0
    w = valid / np.maximum(deg, 1.0)[None, :]
    return w.astype(np.float32)


_WSTEN = _edge_tables()


def _sc_mean(x0p):
    info = plsc.get_sparse_core_info()
    nc, ns = info.num_cores, info.num_subcores
    cpw = _C // (nc * ns)
    mesh = plsc.VectorSubcoreMesh(core_axis_name="c", subcore_axis_name="s")

    @functools.partial(
        pl.kernel,
        mesh=mesh,
        out_type=jax.ShapeDtypeStruct((_C * _NROW,), jnp.float32),
        scratch_types=[
            pltpu.VMEM((_PADL + _NPAD,), jnp.float32),
            pltpu.VMEM((_NPAD,), jnp.float32),
            pltpu.VMEM((4 * _NPAD,), jnp.float32),
        ],
    )
    def k(x_hbm, w_hbm, out_hbm, row_in, row_out, w_vm):
        wid = lax.axis_index("s") * nc + lax.axis_index("c")
        pltpu.sync_copy(w_hbm, w_vm)
        zeros = jnp.zeros((16,), jnp.float32)
        for z in range(_PADL // 16):
            row_in[pl.ds(z * 16, 16)] = zeros
        for cc in range(cpw):
            ch = wid * cpw + cc
            pltpu.sync_copy(x_hbm.at[pl.ds(ch * _NROW, _NROW)],
                            row_in.at[pl.ds(_PADL, _NROW)])

            def step(j, carry):
                s = j * 16
                acc = row_in[pl.ds(_PADL + s - _OFFS[0], 16)] * w_vm[pl.ds(s, 16)]
                acc += row_in[pl.ds(_PADL + s - _OFFS[1], 16)] * w_vm[pl.ds(_NPAD + s, 16)]
                acc += row_in[pl.ds(_PADL + s - _OFFS[2], 16)] * w_vm[pl.ds(2 * _NPAD + s, 16)]
                acc += row_in[pl.ds(_PADL + s - _OFFS[3], 16)] * w_vm[pl.ds(3 * _NPAD + s, 16)]
                row_out[pl.ds(s, 16)] = acc
                return carry

            lax.fori_loop(0, _NPAD // 16, step, 0)
            pltpu.sync_copy(row_out, out_hbm.at[pl.ds(ch * _NROW, _NROW)])

    return k(x0p.reshape(_C * _NROW), jnp.asarray(_WSTEN.reshape(4 * _NPAD)))


def _tc_body(xcp_ref, wrt_ref, bl_ref, mean_ref, wlc_ref, out_ref, m_scr):
    b = pl.program_id(0)

    @pl.when(b == 0)
    def _():
        m = mean_ref[...][:, :_NN]
        mm = lax.dot_general(
            m, wlc_ref[...], (((0,), (0,)), ((), ())),
            preferred_element_type=jnp.float32,
        )
        for l2 in range(_NL):
            m_scr[l2] = mm[:, l2 * _C:(l2 + 1) * _C]

    xn = xcp_ref[0]
    for l in range(_NL):
        y = lax.dot_general(
            xn, wrt_ref[l], (((1,), (0,)), ((), ())),
            preferred_element_type=jnp.float32,
        )
        y = y + bl_ref[l][None, :]
        y = y + jnp.where(b == 0, m_scr[l], jnp.zeros_like(y))
        out_ref[0, l] = jnp.where(y > 0.0, y, jnp.exp(y) - 1.0)


def _tc_forward(xcp, wr_t, b_l, mean_p, wl_cat):
    bsz = xcp.shape[0]
    return pl.pallas_call(
        _tc_body,
        grid=(bsz,),
        in_specs=[
            pl.BlockSpec((1, _NN, _C), lambda b: (b, 0, 0)),
            pl.BlockSpec((_NL, _C, _C), lambda b: (0, 0, 0)),
            pl.BlockSpec((_NL, _C), lambda b: (0, 0)),
            pl.BlockSpec((_C, _NROW), lambda b: (0, 0)),
            pl.BlockSpec((_C, _NL * _C), lambda b: (0, 0)),
        ],
        out_specs=pl.BlockSpec((1, _NL, _NN, _C), lambda b: (b, 0, 0, 0)),
        out_shape=jax.ShapeDtypeStruct((bsz, _NL, _NN, _C), jnp.float32),
        scratch_shapes=[pltpu.VMEM((_NL, _NN, _C), jnp.float32)],
        compiler_params=pltpu.CompilerParams(
            dimension_semantics=("arbitrary",),
        ),
    )(xcp, wr_t, b_l, mean_p, wl_cat)


def kernel(x, W_l, b_l, W_r):
    bsz, C, H, W = x.shape
    xc = x.reshape(bsz, C, _NN)
    xn = jnp.swapaxes(xc, 1, 2)
    x0p = jnp.pad(xc[0], ((0, 0), (0, _NROW - _NN)))
    mean_p = _sc_mean(x0p).reshape(C, _NROW)
    wr_t = jnp.transpose(W_r, (0, 2, 1))
    wl_cat = jnp.transpose(W_l, (2, 0, 1)).reshape(C, _NL * C)
    out = _tc_forward(xn, wr_t, b_l, mean_p, wl_cat)
    z = out.reshape(bsz, _NL * C, _NN)
    xr = x.reshape(bsz, C, _NN)
    return jnp.concatenate([z, xr], axis=1).reshape(bsz, (_NL + 1) * C, H, W)
